# baseline (speedup 1.0000x reference)
import functools

import jax
import jax.numpy as jnp
from jax import lax
from jax.experimental import pallas as pl
from jax.experimental.pallas import tpu as pltpu
from jax.experimental.pallas import tpu_sc as plsc

_INFO = plsc.get_sparse_core_info()
_NC = _INFO.num_cores
_NS = _INFO.num_subcores
_NW = _NC * _NS

_T = 50
_BT = 16384
_D = 64
_V2 = 500000
_BW = _BT // _NW
_CB = 128
_SB = _BW // _CB
_NCH = _T * _SB
_NBUF = 4
_LOOK = 2

_mesh = plsc.VectorSubcoreMesh(core_axis_name="c", subcore_axis_name="s")


@functools.partial(
    pl.kernel,
    out_type=jax.ShapeDtypeStruct((_T, _D, _BT), jnp.float32),
    mesh=_mesh,
    compiler_params=pltpu.CompilerParams(needs_layout_passes=False),
    scratch_types=[
        pltpu.VMEM((_T, _BW), jnp.int32),
        pltpu.VMEM((_NBUF, _CB), jnp.int32),
        pltpu.VMEM((_NBUF, _CB, 128), jnp.float32),
        pltpu.VMEM((_NBUF, _D, _CB), jnp.float32),
    ] + [pltpu.SemaphoreType.DMA] * (2 * _NBUF),
)
def _embed_kernel(xt_hbm, table_hbm, out_hbm, idx_v, row_v, rows_v, tbuf_v,
                  *sems):
    gsem = sems[:_NBUF]
    ssem = sems[_NBUF:]
    wid = lax.axis_index("s") * _NC + lax.axis_index("c")
    b0 = wid * _BW

    pltpu.sync_copy(xt_hbm.at[:, pl.ds(b0, _BW)], idx_v)

    iota = lax.broadcasted_iota(jnp.int32, (16,), 0)

    def fire_gather(ch, b):
        t = ch // _SB
        sb = lax.rem(ch, _SB)
        for j in range(_CB // 16):
            v = idx_v[t, pl.ds(sb * _CB + 16 * j, 16)]
            row_v[b, pl.ds(16 * j, 16)] = lax.shift_right_logical(v, 1)
        pltpu.async_copy(table_hbm.at[row_v.at[b]], rows_v.at[b], gsem[b])

    def drain_gather(b):
        pltpu.make_async_copy(
            table_hbm.at[row_v.at[b]], rows_v.at[b], gsem[b]).wait()

    def fire_store(ch, b):
        t = ch // _SB
        sb = lax.rem(ch, _SB)
        pltpu.async_copy(
            tbuf_v.at[b],
            out_hbm.at[t, :, pl.ds(b0 + sb * _CB, _CB)],
            ssem[b])

    def drain_store(ch, b):
        t = ch // _SB
        sb = lax.rem(ch, _SB)
        pltpu.make_async_copy(
            tbuf_v.at[b],
            out_hbm.at[t, :, pl.ds(b0 + sb * _CB, _CB)],
            ssem[b]).wait()

    def transpose_chunk(ch, b):
        t = ch // _SB
        sb = lax.rem(ch, _SB)
        offs = []
        for j in range(_CB // 16):
            v = idx_v[t, pl.ds(sb * _CB + 16 * j, 16)]
            offs.append(lax.shift_left(jnp.bitwise_and(v, 1), 6))

        rowids = [iota + 16 * j for j in range(_CB // 16)]

        def dhalf(dd, _):
            for du in range(_D // 2):
                d = dd * (_D // 2) + du
                for j in range(_CB // 16):
                    col = plsc.load_gather(
                        rows_v.at[b], [rowids[j], offs[j] + d])
                    tbuf_v[b, d, pl.ds(16 * j, 16)] = col
            return ()

        lax.fori_loop(0, 2, dhalf, ())

    for ch in range(_LOOK):
        fire_gather(ch, ch)

    def body(i, _):
        for bb in range(_NBUF):
            g = i * _NBUF + bb
            drain_gather(bb)

            @pl.when(g >= _NBUF)
            def _():
                drain_store(g - _NBUF, bb)

            transpose_chunk(g, bb)
            fire_store(g, bb)

            @pl.when(g + _LOOK < _NCH)
            def _():
                fire_gather(g + _LOOK, (bb + _LOOK) % _NBUF)
        return ()

    lax.fori_loop(0, _NCH // _NBUF, body, ())

    for ch in range(_NCH - _NBUF, _NCH):
        drain_store(ch, ch % _NBUF)


def kernel(x, embeddings):
    table128 = embeddings.reshape(_V2, 128)
    res = _embed_kernel(x.T, table128)
    return jnp.transpose(res, (2, 0, 1))


# speedup vs baseline: 1.4684x; 1.4684x over previous
import functools

import jax
import jax.numpy as jnp
from jax import lax
from jax.experimental import pallas as pl
from jax.experimental.pallas import tpu as pltpu
from jax.experimental.pallas import tpu_sc as plsc

_INFO = plsc.get_sparse_core_info()
_NC = _INFO.num_cores
_NS = _INFO.num_subcores
_NW = _NC * _NS

_B = 16384 * 50
_D = 64
_IW = 128
_RPC = 2
_CROWS = _RPC * _IW
_B_PER_W = _B // _NW
_IDX_ROWS = _B_PER_W // _IW
_NCH = _B_PER_W // _CROWS
_NBUF = 3
_LOOK = 2

_mesh = plsc.VectorSubcoreMesh(core_axis_name="c", subcore_axis_name="s")


@functools.partial(
    pl.kernel,
    out_type=jax.ShapeDtypeStruct((_B, _D), jnp.float32),
    mesh=_mesh,
    compiler_params=pltpu.CompilerParams(use_tc_tiling_on_sc=False),
    scratch_types=[
        pltpu.VMEM((_IDX_ROWS, _IW), jnp.int32),
        pltpu.VMEM((_NBUF, _CROWS, _D), jnp.float32),
        pltpu.SemaphoreType.DMA,
        pltpu.SemaphoreType.DMA,
        pltpu.SemaphoreType.DMA,
        pltpu.SemaphoreType.DMA,
        pltpu.SemaphoreType.DMA,
        pltpu.SemaphoreType.DMA,
    ],
)
def _embed_kernel(idx_hbm, table_hbm, out_hbm, idx_v, rows_v,
                  g0, g1, g2, s0, s1, s2):
    gsem = (g0, g1, g2)
    ssem = (s0, s1, s2)
    wid = lax.axis_index("s") * _NC + lax.axis_index("c")
    row_base = wid * _IDX_ROWS
    out_base = wid * _B_PER_W

    pltpu.sync_copy(idx_hbm.at[pl.ds(row_base, _IDX_ROWS)], idx_v)

    def fire_gather(ch, b):
        for r in range(_RPC):
            pltpu.async_copy(
                table_hbm.at[idx_v.at[ch * _RPC + r]],
                rows_v.at[b].at[pl.ds(r * _IW, _IW)],
                gsem[b],
            )

    def drain_gather(ch, b):
        for r in range(_RPC):
            pltpu.make_async_copy(
                table_hbm.at[idx_v.at[ch * _RPC + r]],
                rows_v.at[b].at[pl.ds(r * _IW, _IW)],
                gsem[b],
            ).wait()

    def fire_store(ch, b):
        pltpu.async_copy(
            rows_v.at[b], out_hbm.at[pl.ds(out_base + ch * _CROWS, _CROWS)],
            ssem[b],
        )

    def drain_store(ch, b):
        pltpu.make_async_copy(
            rows_v.at[b], out_hbm.at[pl.ds(out_base + ch * _CROWS, _CROWS)],
            ssem[b],
        ).wait()

    for ch in range(_LOOK):
        fire_gather(ch, ch % _NBUF)

    def body(i, _):
        for bb in range(_NBUF):
            g = i * _NBUF + bb
            drain_gather(g, bb)
            fire_store(g, bb)
            nb = (bb + _LOOK) % _NBUF

            @pl.when(jnp.logical_and(g >= 1, g + _LOOK < _NCH))
            def _():
                drain_store(g - 1, nb)

            @pl.when(g + _LOOK < _NCH)
            def _():
                fire_gather(g + _LOOK, nb)
        return ()

    lax.fori_loop(0, _NCH // _NBUF, body, ())

    g = _NCH - 1
    bb = g % _NBUF
    drain_gather(g, bb)
    fire_store(g, bb)

    for ch in range(_NCH - 3, _NCH):
        drain_store(ch, ch % _NBUF)


def kernel(x, embeddings):
    idx2d = x.reshape(_B // _IW, _IW).astype(jnp.int32)
    out = _embed_kernel(idx2d, embeddings)
    return out.reshape(x.shape[0], x.shape[1], _D)
